# baseline (speedup 1.0000x reference)
import jax
import jax.numpy as jnp
from jax import lax
from jax.experimental import pallas as pl
from jax.experimental.pallas import tpu as pltpu
from jax.experimental.pallas import tpu_sc as plsc

_NF = 26
_ROWS = 100001
_DIM = 16
_B = 16384
_NC = 2
_NS = 16
_NW = _NC * _NS
_BPW = _B // _NW
_LANES = 16
_GCH = 128


def _emb_body(idx_hbm, tab_hbm, out_hbm, idx_v, rows_v, acc_v, sem):
    wid = lax.axis_index("s") * _NC + lax.axis_index("c")
    base = wid * _BPW

    pltpu.sync_copy(idx_hbm.at[:, pl.ds(base, _BPW)], idx_v)

    for f in range(1, _NF):
        off = jnp.full((_LANES,), f * _ROWS, jnp.int32)

        def _add_off(j, carry, f=f, off=off):
            s = pl.ds(j * _LANES, _LANES)
            idx_v[f, s] = idx_v[f, s] + off
            return carry

        lax.fori_loop(0, _BPW // _LANES, _add_off, 0)

    nch = _BPW // _GCH
    for f in range(_NF):
        copies = [
            pltpu.async_copy(
                tab_hbm.at[idx_v.at[f, pl.ds(c * _GCH, _GCH)]],
                rows_v.at[pl.ds(c * _GCH, _GCH)],
                sem,
            )
            for c in range(nch)
        ]
        for cp in copies:
            cp.wait()

        if f == 0:
            def _init(i, carry):
                acc_v[i] = rows_v[i]
                return carry

            lax.fori_loop(0, _BPW, _init, 0)
        else:
            def _accum(i, carry):
                acc_v[i] = acc_v[i] + rows_v[i]
                return carry

            lax.fori_loop(0, _BPW, _accum, 0)

    inv = jnp.full((_LANES,), 1.0 / _NF, jnp.float32)

    def _scale(i, carry):
        acc_v[i] = acc_v[i] * inv
        return carry

    lax.fori_loop(0, _BPW, _scale, 0)
    pltpu.sync_copy(acc_v, out_hbm.at[pl.ds(base, _BPW)])


def kernel(inputs, batch_size, tables):
    del batch_size
    idx_t = inputs.T
    tab_flat = tables.reshape(_NF * _ROWS, _DIM)
    run = pl.kernel(
        _emb_body,
        out_type=jax.ShapeDtypeStruct((_B, _DIM), jnp.float32),
        mesh=plsc.VectorSubcoreMesh(core_axis_name="c", subcore_axis_name="s"),
        scratch_types=[
            pltpu.VMEM((_NF, _BPW), jnp.int32),
            pltpu.VMEM((_BPW, _DIM), jnp.float32),
            pltpu.VMEM((_BPW, _DIM), jnp.float32),
            pltpu.SemaphoreType.DMA,
        ],
        compiler_params=pltpu.CompilerParams(use_tc_tiling_on_sc=False),
    )
    return run(idx_t, tab_flat)


# speedup vs baseline: 1.9657x; 1.9657x over previous
import jax
import jax.numpy as jnp
from jax import lax
from jax.experimental import pallas as pl
from jax.experimental.pallas import tpu as pltpu
from jax.experimental.pallas import tpu_sc as plsc

_NF = 26
_ROWS = 100001
_DIM = 16
_B = 16384
_NC = 2
_NS = 16
_NW = _NC * _NS
_BPW = _B // _NW
_LANES = 16
_GCH = 128


def _emb_body(idx_hbm, tab_hbm, out_hbm, idx2d_v, col_v, rows_v, acc_v, sem):
    wid = lax.axis_index("s") * _NC + lax.axis_index("c")
    base = wid * _BPW

    pltpu.sync_copy(idx_hbm.at[pl.ds(base, _BPW), :], idx2d_v)

    lanes = lax.iota(jnp.int32, _LANES)
    nch = _BPW // _GCH
    for f in range(_NF):
        colf = jnp.full((_LANES,), f, jnp.int32)

        def _extract(j, carry, colf=colf):
            r = lanes + j * _LANES
            vals = plsc.load_gather(idx2d_v, [r, colf])
            col_v[pl.ds(j * _LANES, _LANES)] = vals
            return carry

        lax.fori_loop(0, _BPW // _LANES, _extract, 0, unroll=4)

        copies = [
            pltpu.async_copy(
                tab_hbm.at[f].at[col_v.at[pl.ds(c * _GCH, _GCH)]],
                rows_v.at[pl.ds(c * _GCH, _GCH)],
                sem,
            )
            for c in range(nch)
        ]
        for cp in copies:
            cp.wait()

        if f == 0:
            def _init(i, carry):
                acc_v[i] = rows_v[i]
                return carry

            lax.fori_loop(0, _BPW, _init, 0, unroll=8)
        else:
            def _accum(i, carry):
                acc_v[i] = acc_v[i] + rows_v[i]
                return carry

            lax.fori_loop(0, _BPW, _accum, 0, unroll=8)

    inv = jnp.full((_LANES,), 1.0 / _NF, jnp.float32)

    def _scale(i, carry):
        acc_v[i] = acc_v[i] * inv
        return carry

    lax.fori_loop(0, _BPW, _scale, 0, unroll=8)
    pltpu.sync_copy(acc_v, out_hbm.at[pl.ds(base, _BPW)])


def kernel(inputs, batch_size, tables):
    del batch_size
    run = pl.kernel(
        _emb_body,
        out_type=jax.ShapeDtypeStruct((_B, _DIM), jnp.float32),
        mesh=plsc.VectorSubcoreMesh(core_axis_name="c", subcore_axis_name="s"),
        scratch_types=[
            pltpu.VMEM((_BPW, _NF), jnp.int32),
            pltpu.VMEM((_BPW,), jnp.int32),
            pltpu.VMEM((_BPW, _DIM), jnp.float32),
            pltpu.VMEM((_BPW, _DIM), jnp.float32),
            pltpu.SemaphoreType.DMA,
        ],
        compiler_params=pltpu.CompilerParams(
            use_tc_tiling_on_sc=False, needs_layout_passes=False
        ),
    )
    return run(inputs, tables)


# speedup vs baseline: 3.1945x; 1.6251x over previous
import jax
import jax.numpy as jnp
from jax import lax
from jax.experimental import pallas as pl
from jax.experimental.pallas import tpu as pltpu
from jax.experimental.pallas import tpu_sc as plsc

_NF = 26
_VOC = 100001
_VPAD = 100008
_NLINE = _VPAD // 8
_DIM = 16
_B = 16384
_NC = 2
_NS = 16
_NW = _NC * _NS
_BPW = _B // _NW
_LANES = 16
_GCH = 128


def _emb_body(idx_hbm, tab_hbm, out_hbm, idx_v, g_v, s_v, rows_v, acc_v, sem):
    wid = lax.axis_index("s") * _NC + lax.axis_index("c")
    base = wid * _BPW

    for f in range(_NF):
        pltpu.sync_copy(idx_hbm.at[f, pl.ds(base, _BPW)], idx_v.at[pl.ds(f * _BPW, _BPW)])

    seven = jnp.full((_LANES,), 7, jnp.int32)
    nchunk = _BPW // _LANES
    nch = _BPW // _GCH
    inv = jnp.full((_LANES,), 1.0 / _NF, jnp.float32)

    for f in range(_NF):
        def _split(j, carry, f=f):
            s = pl.ds(j * _LANES, _LANES)
            v = idx_v[pl.ds(f * _BPW + j * _LANES, _LANES)]
            g_v[s] = lax.shift_right_logical(v, 3)
            s_v[s] = lax.shift_left(lax.bitwise_and(v, seven), 4)
            return carry

        lax.fori_loop(0, nchunk, _split, 0)

        copies = [
            pltpu.async_copy(
                tab_hbm.at[f].at[g_v.at[pl.ds(c * _GCH, _GCH)]],
                rows_v.at[pl.ds(c * _GCH, _GCH)],
                sem,
            )
            for c in range(nch)
        ]
        for cp in copies:
            cp.wait()

        lanes = lax.iota(jnp.int32, _LANES)

        if f == 0:
            def _ext0(j, carry):
                rows16 = lanes + j * _LANES
                cols = s_v[pl.ds(j * _LANES, _LANES)]
                for d in range(_DIM):
                    acc_v[d, pl.ds(j * _LANES, _LANES)] = plsc.load_gather(
                        rows_v, [rows16, cols + d]
                    )
                return carry

            lax.fori_loop(0, nchunk, _ext0, 0)
        else:
            def _ext(j, carry):
                rows16 = lanes + j * _LANES
                cols = s_v[pl.ds(j * _LANES, _LANES)]
                for d in range(_DIM):
                    sl = pl.ds(j * _LANES, _LANES)
                    acc_v[d, sl] = acc_v[d, sl] + plsc.load_gather(
                        rows_v, [rows16, cols + d]
                    )
                return carry

            lax.fori_loop(0, nchunk, _ext, 0)

    def _scale(j, carry):
        for d in range(_DIM):
            sl = pl.ds(j * _LANES, _LANES)
            acc_v[d, sl] = acc_v[d, sl] * inv
        return carry

    lax.fori_loop(0, nchunk, _scale, 0)
    pltpu.sync_copy(acc_v, out_hbm.at[:, pl.ds(base, _BPW)])


def kernel(inputs, batch_size, tables):
    del batch_size
    idx_t = inputs.T
    tab_p = jnp.pad(tables, ((0, 0), (0, _VPAD - _VOC), (0, 0)))
    tab_l = tab_p.reshape(_NF, _NLINE, 128)
    run = pl.kernel(
        _emb_body,
        out_type=jax.ShapeDtypeStruct((_DIM, _B), jnp.float32),
        mesh=plsc.VectorSubcoreMesh(core_axis_name="c", subcore_axis_name="s"),
        scratch_types=[
            pltpu.VMEM((_NF * _BPW,), jnp.int32),
            pltpu.VMEM((_BPW,), jnp.int32),
            pltpu.VMEM((_BPW,), jnp.int32),
            pltpu.VMEM((_BPW, 128), jnp.float32),
            pltpu.VMEM((_DIM, _BPW), jnp.float32),
            pltpu.SemaphoreType.DMA,
        ],
        compiler_params=pltpu.CompilerParams(
            use_tc_tiling_on_sc=True, needs_layout_passes=False
        ),
    )
    out_t = run(idx_t, tab_l)
    return out_t.T


# speedup vs baseline: 4.5746x; 1.4320x over previous
import jax
import jax.numpy as jnp
from jax import lax
from jax.experimental import pallas as pl
from jax.experimental.pallas import tpu as pltpu
from jax.experimental.pallas import tpu_sc as plsc

_NF = 26
_VOC = 100001
_DIM = 16
_B = 16384
_NC = 2
_NS = 16
_NW = _NC * _NS
_BPW = _B // _NW
_LANES = 16
_GCH = 128

_RC = 640
_LPB = _RC // 8
_NBLK = _VOC // _RC
_NLINE = _NBLK * _LPB
_TAIL0 = _NLINE * 8
_TROWS = 32


def _repack_body(tab_hbm, pk_hbm, stage_v, outb_v, sem):
    wid = lax.axis_index("s") * _NC + lax.axis_index("c")
    lanes = lax.iota(jnp.int32, _LANES)

    for f in range(_NF):
        def _block(it, carry, f=f):
            bi = wid + it * _NW

            @pl.when(bi < _NBLK)
            def _do():
                rc = pl.multiple_of(bi * _RC, 128)
                pltpu.sync_copy(tab_hbm.at[f, :, pl.ds(rc, _RC)], stage_v)

                def _line(gl, carry2):
                    for j in range(8):
                        col = gl * 8 + j
                        vals = plsc.load_gather(
                            stage_v,
                            [lanes, jnp.full((_LANES,), 0, jnp.int32) + col],
                        )
                        outb_v[gl, pl.ds(j * _LANES, _LANES)] = vals
                    return carry2

                lax.fori_loop(0, _LPB, _line, 0)
                g = pl.multiple_of(bi * _LPB, 8)
                pltpu.sync_copy(outb_v, pk_hbm.at[f, pl.ds(g, _LPB), :])

            return carry

        lax.fori_loop(0, 5, _block, 0)


def _lookup_body(idx_hbm, pk_hbm, tail_hbm, out_hbm,
                 idx_v, g_v, gs_v, s_v, rows_v, acc_v, sem):
    wid = lax.axis_index("s") * _NC + lax.axis_index("c")
    base = wid * _BPW

    seven = jnp.full((_LANES,), 7, jnp.int32)
    gmax = jnp.full((_LANES,), _NLINE - 1, jnp.int32)
    nchunk = _BPW // _LANES
    nch = _BPW // _GCH
    inv = jnp.full((_LANES,), 1.0 / _NF, jnp.float32)
    lanes = lax.iota(jnp.int32, _LANES)

    for f in range(_NF):
        pltpu.sync_copy(idx_hbm.at[f, pl.ds(base, _BPW)], idx_v)
        pltpu.sync_copy(tail_hbm.at[f], rows_v.at[pl.ds(_BPW, _TROWS)])

        def _split(j, carry):
            s = pl.ds(j * _LANES, _LANES)
            v = idx_v[s]
            g = lax.shift_right_logical(v, 3)
            g_v[s] = g
            gs_v[s] = jnp.minimum(g, gmax)
            s_v[s] = lax.shift_left(lax.bitwise_and(v, seven), 4)
            return carry

        lax.fori_loop(0, nchunk, _split, 0)

        copies = [
            pltpu.async_copy(
                pk_hbm.at[f].at[gs_v.at[pl.ds(c * _GCH, _GCH)]],
                rows_v.at[pl.ds(c * _GCH, _GCH)],
                sem,
            )
            for c in range(nch)
        ]
        for cp in copies:
            cp.wait()

        if f == 0:
            def _ext0(j, carry):
                sl = pl.ds(j * _LANES, _LANES)
                g16 = g_v[sl]
                rows16 = jnp.where(
                    g16 > gmax, g16 + (_BPW - _NLINE), lanes + j * _LANES
                )
                cols = s_v[sl]
                for d in range(_DIM):
                    acc_v[d, sl] = plsc.load_gather(rows_v, [rows16, cols + d])
                return carry

            lax.fori_loop(0, nchunk, _ext0, 0)
        else:
            def _ext(j, carry):
                sl = pl.ds(j * _LANES, _LANES)
                g16 = g_v[sl]
                rows16 = jnp.where(
                    g16 > gmax, g16 + (_BPW - _NLINE), lanes + j * _LANES
                )
                cols = s_v[sl]
                for d in range(_DIM):
                    acc_v[d, sl] = acc_v[d, sl] + plsc.load_gather(
                        rows_v, [rows16, cols + d]
                    )
                return carry

            lax.fori_loop(0, nchunk, _ext, 0)

    def _scale(j, carry):
        for d in range(_DIM):
            sl = pl.ds(j * _LANES, _LANES)
            acc_v[d, sl] = acc_v[d, sl] * inv
        return carry

    lax.fori_loop(0, nchunk, _scale, 0)
    pltpu.sync_copy(acc_v, out_hbm.at[:, pl.ds(base, _BPW)])


def kernel(inputs, batch_size, tables):
    del batch_size
    idx_t = inputs.T
    tab_t = jnp.transpose(tables, (0, 2, 1))
    tab_tail = jnp.pad(
        lax.slice(tables, (0, _TAIL0, 0), (_NF, _VOC, _DIM)),
        ((0, 0), (0, _TROWS * 8 - (_VOC - _TAIL0)), (0, 0)),
    ).reshape(_NF, _TROWS, 128)

    repack = pl.kernel(
        _repack_body,
        out_type=jax.ShapeDtypeStruct((_NF, _NLINE, 128), jnp.float32),
        mesh=plsc.VectorSubcoreMesh(core_axis_name="c", subcore_axis_name="s"),
        scratch_types=[
            pltpu.VMEM((_DIM, _RC), jnp.float32),
            pltpu.VMEM((_LPB, 128), jnp.float32),
            pltpu.SemaphoreType.DMA,
        ],
        compiler_params=pltpu.CompilerParams(
            use_tc_tiling_on_sc=True, needs_layout_passes=False
        ),
    )
    packed = repack(tab_t)

    lookup = pl.kernel(
        _lookup_body,
        out_type=jax.ShapeDtypeStruct((_DIM, _B), jnp.float32),
        mesh=plsc.VectorSubcoreMesh(core_axis_name="c", subcore_axis_name="s"),
        scratch_types=[
            pltpu.VMEM((_BPW,), jnp.int32),
            pltpu.VMEM((_BPW,), jnp.int32),
            pltpu.VMEM((_BPW,), jnp.int32),
            pltpu.VMEM((_BPW,), jnp.int32),
            pltpu.VMEM((_BPW + _TROWS, 128), jnp.float32),
            pltpu.VMEM((_DIM, _BPW), jnp.float32),
            pltpu.SemaphoreType.DMA,
        ],
        compiler_params=pltpu.CompilerParams(
            use_tc_tiling_on_sc=True, needs_layout_passes=False
        ),
    )
    out_t = lookup(idx_t, packed, tab_tail)
    return out_t.T
